# baseline (speedup 1.0000x reference)
import jax
import jax.numpy as jnp
from jax import lax
from jax.experimental import pallas as pl
from jax.experimental.pallas import tpu as pltpu
from jax.experimental.pallas import tpu_sc as plsc

_NC = 2
_NS = 16
_NW = _NC * _NS
_L = 16
_D = 64
_B = 16384
_BPW = _B // _NW
_DEPTH = 4


_TAIL = 999936
_LASTC = 999808


def _sc_body(user_hbm, item_hbm, uT_hbm, vT_hbm, tu_hbm, tv_hbm, out_hbm,
             idx_u, idx_v, bu, bv, tail_u, tail_v, p_buf, out_v,
             s0, s1, s2, s3):
    wid = lax.axis_index("s") * _NC + lax.axis_index("c")
    base = wid * _BPW
    sems = [s0, s1, s2, s3]

    pltpu.sync_copy(user_hbm.at[pl.ds(base, _BPW)], idx_u)
    pltpu.sync_copy(item_hbm.at[pl.ds(base, _BPW)], idx_v)
    pltpu.sync_copy(tu_hbm, tail_u)
    pltpu.sync_copy(tv_hbm, tail_v)

    def issue(iu, iv, slot, sem):
        cu = pl.multiple_of(
            jnp.minimum(lax.shift_right_logical(iu, 7) * 128, _LASTC), 128)
        cv = pl.multiple_of(
            jnp.minimum(lax.shift_right_logical(iv, 7) * 128, _LASTC), 128)
        pltpu.async_copy(uT_hbm.at[:, pl.ds(cu, 128)], bu.at[slot], sem)
        pltpu.async_copy(vT_hbm.at[:, pl.ds(cv, 128)], bv.at[slot], sem)

    def drain(slot, sem):
        pltpu.make_async_copy(uT_hbm.at[:, pl.ds(0, 128)], bu.at[slot], sem).wait()
        pltpu.make_async_copy(vT_hbm.at[:, pl.ds(0, 128)], bv.at[slot], sem).wait()

    lane = lax.iota(jnp.int32, _L)
    col_idx = lane * _L
    d_vecs = [lane + g * _L for g in range(_D // _L)]

    iu0 = idx_u[pl.ds(0, _L)]
    iv0 = idx_v[pl.ds(0, _L)]
    for k in range(_DEPTH):
        issue(iu0[k], iv0[k], k, sems[k])

    def step(j, carry):
        p = j * _L
        iu_vec = idx_u[pl.ds(p, _L)]
        iv_vec = idx_v[pl.ds(p, _L)]
        pn = jnp.minimum(p + _L, _BPW - _L)
        iu_nxt = idx_u[pl.ds(pn, _L)]
        iv_nxt = idx_v[pl.ds(pn, _L)]
        for k in range(_L):
            slot = k % _DEPTH
            drain(slot, sems[slot])
            iu = iu_vec[k]
            iv = iv_vec[k]
            lu = jnp.full((_L,), 0, jnp.int32) + lax.bitwise_and(iu, 127)
            lv = jnp.full((_L,), 0, jnp.int32) + lax.bitwise_and(iv, 127)
            tmu = jnp.full((_L,), iu >= _TAIL)
            tmv = jnp.full((_L,), iv >= _TAIL)
            tlu = jnp.full((_L,), 0, jnp.int32) + jnp.maximum(iu - _TAIL, 0)
            tlv = jnp.full((_L,), 0, jnp.int32) + jnp.maximum(iv - _TAIL, 0)
            acc = jnp.zeros((_L,), jnp.float32)
            for g in range(_D // _L):
                uvals = jnp.where(tmu,
                                  plsc.load_gather(tail_u, [d_vecs[g], tlu]),
                                  plsc.load_gather(bu.at[slot], [d_vecs[g], lu]))
                vvals = jnp.where(tmv,
                                  plsc.load_gather(tail_v, [d_vecs[g], tlv]),
                                  plsc.load_gather(bv.at[slot], [d_vecs[g], lv]))
                acc += uvals * vvals
            p_buf[pl.ds(k * _L, _L)] = acc
            ahead = k + _DEPTH
            @pl.when(p + ahead < _BPW)
            def _():
                if ahead < _L:
                    issue(iu_vec[ahead], iv_vec[ahead], slot, sems[slot])
                else:
                    issue(iu_nxt[ahead - _L], iv_nxt[ahead - _L], slot, sems[slot])
        tot = plsc.load_gather(p_buf, [col_idx])
        for t in range(1, _L):
            tot += plsc.load_gather(p_buf, [col_idx + t])
        out_v[pl.ds(p, _L)] = tot
        return carry

    lax.fori_loop(0, _BPW // _L, step, 0)

    pltpu.sync_copy(out_v, out_hbm.at[pl.ds(base, _BPW)])


@jax.jit
def kernel(user, item, user_factors, item_factors):
    order = jnp.argsort(user)
    user_s = user[order]
    item_s = item[order]
    uT = user_factors.T
    vT = item_factors.T
    tu = user_factors[_TAIL:].T
    tv = item_factors[_TAIL:].T
    mesh = plsc.VectorSubcoreMesh(core_axis_name="c", subcore_axis_name="s",
                                  num_cores=_NC, num_subcores=_NS)
    run = pl.kernel(
        _sc_body,
        out_type=jax.ShapeDtypeStruct((_B,), jnp.float32),
        mesh=mesh,
        scratch_types=[
            pltpu.VMEM((_BPW,), jnp.int32),
            pltpu.VMEM((_BPW,), jnp.int32),
            pltpu.VMEM((_DEPTH, _D, 128), jnp.float32),
            pltpu.VMEM((_DEPTH, _D, 128), jnp.float32),
            pltpu.VMEM((_D, _D), jnp.float32),
            pltpu.VMEM((_D, _D), jnp.float32),
            pltpu.VMEM((_L * _L,), jnp.float32),
            pltpu.VMEM((_BPW,), jnp.float32),
            pltpu.SemaphoreType.DMA,
            pltpu.SemaphoreType.DMA,
            pltpu.SemaphoreType.DMA,
            pltpu.SemaphoreType.DMA,
        ],
        compiler_params=pltpu.CompilerParams(needs_layout_passes=False,
                                             use_tc_tiling_on_sc=True),
    )
    out_s = run(user_s, item_s, uT, vT, tu, tv)
    return jnp.zeros((_B,), jnp.float32).at[order].set(out_s)


# speedup vs baseline: 1.1741x; 1.1741x over previous
import jax
import jax.numpy as jnp
from jax import lax
from jax.experimental import pallas as pl
from jax.experimental.pallas import tpu as pltpu
from jax.experimental.pallas import tpu_sc as plsc

_NC = 2
_NS = 16
_NW = _NC * _NS
_L = 16
_D = 64
_B = 16384
_BPW = _B // _NW
_DEPTH = 4


_TAIL = 999936
_LASTC = 999808


def _sc_body(user_hbm, item_hbm, uT_hbm, vT_hbm, tu_hbm, tv_hbm, out_hbm,
             idx_u, idx_v, bu, bv, tail_u, tail_v, p_buf, out_v,
             s0, s1, s2, s3):
    wid = lax.axis_index("s") * _NC + lax.axis_index("c")
    base = wid * _BPW
    sems = [s0, s1, s2, s3]

    pltpu.sync_copy(user_hbm.at[pl.ds(base, _BPW)], idx_u)
    pltpu.sync_copy(item_hbm.at[pl.ds(base, _BPW)], idx_v)
    pltpu.sync_copy(tu_hbm, tail_u)
    pltpu.sync_copy(tv_hbm, tail_v)

    def issue(iu, iv, slot, sem):
        cu = pl.multiple_of(
            jnp.minimum(lax.shift_right_logical(iu, 7) * 128, _LASTC), 128)
        cv = pl.multiple_of(
            jnp.minimum(lax.shift_right_logical(iv, 7) * 128, _LASTC), 128)
        pltpu.async_copy(uT_hbm.at[:, pl.ds(cu, 128)], bu.at[slot], sem)
        pltpu.async_copy(vT_hbm.at[:, pl.ds(cv, 128)], bv.at[slot], sem)

    def drain(slot, sem):
        pltpu.make_async_copy(uT_hbm.at[:, pl.ds(0, 128)], bu.at[slot], sem).wait()
        pltpu.make_async_copy(vT_hbm.at[:, pl.ds(0, 128)], bv.at[slot], sem).wait()

    lane = lax.iota(jnp.int32, _L)
    col_idx = lane * _L
    d_vecs = [lane + g * _L for g in range(_D // _L)]

    iu0 = idx_u[pl.ds(0, _L)]
    iv0 = idx_v[pl.ds(0, _L)]
    for k in range(_DEPTH):
        issue(iu0[k], iv0[k], k, sems[k])

    def step(j, carry):
        p = j * _L
        iu_vec = idx_u[pl.ds(p, _L)]
        iv_vec = idx_v[pl.ds(p, _L)]
        pn = jnp.minimum(p + _L, _BPW - _L)
        iu_nxt = idx_u[pl.ds(pn, _L)]
        iv_nxt = idx_v[pl.ds(pn, _L)]
        for k in range(_L):
            slot = k % _DEPTH
            drain(slot, sems[slot])
            iu = iu_vec[k]
            iv = iv_vec[k]
            lu = jnp.full((_L,), 0, jnp.int32) + lax.bitwise_and(iu, 127)
            lv = jnp.full((_L,), 0, jnp.int32) + lax.bitwise_and(iv, 127)
            tmu = jnp.full((_L,), iu >= _TAIL)
            tmv = jnp.full((_L,), iv >= _TAIL)
            tlu = jnp.full((_L,), 0, jnp.int32) + jnp.maximum(iu - _TAIL, 0)
            tlv = jnp.full((_L,), 0, jnp.int32) + jnp.maximum(iv - _TAIL, 0)
            acc = jnp.zeros((_L,), jnp.float32)
            for g in range(_D // _L):
                uvals = jnp.where(tmu,
                                  plsc.load_gather(tail_u, [d_vecs[g], tlu]),
                                  plsc.load_gather(bu.at[slot], [d_vecs[g], lu]))
                vvals = jnp.where(tmv,
                                  plsc.load_gather(tail_v, [d_vecs[g], tlv]),
                                  plsc.load_gather(bv.at[slot], [d_vecs[g], lv]))
                acc += uvals * vvals
            p_buf[pl.ds(k * _L, _L)] = acc
            ahead = k + _DEPTH
            @pl.when(p + ahead < _BPW)
            def _():
                if ahead < _L:
                    issue(iu_vec[ahead], iv_vec[ahead], slot, sems[slot])
                else:
                    issue(iu_nxt[ahead - _L], iv_nxt[ahead - _L], slot, sems[slot])
        tot = plsc.load_gather(p_buf, [col_idx])
        for t in range(1, _L):
            tot += plsc.load_gather(p_buf, [col_idx + t])
        out_v[pl.ds(p, _L)] = tot
        return carry

    lax.fori_loop(0, _BPW // _L, step, 0)

    pltpu.sync_copy(out_v, out_hbm.at[pl.ds(base, _BPW)])


@jax.jit
def kernel(user, item, user_factors, item_factors):
    uT = user_factors.T
    vT = item_factors.T
    tu = user_factors[_TAIL:].T
    tv = item_factors[_TAIL:].T
    mesh = plsc.VectorSubcoreMesh(core_axis_name="c", subcore_axis_name="s",
                                  num_cores=_NC, num_subcores=_NS)
    run = pl.kernel(
        _sc_body,
        out_type=jax.ShapeDtypeStruct((_B,), jnp.float32),
        mesh=mesh,
        scratch_types=[
            pltpu.VMEM((_BPW,), jnp.int32),
            pltpu.VMEM((_BPW,), jnp.int32),
            pltpu.VMEM((_DEPTH, _D, 128), jnp.float32),
            pltpu.VMEM((_DEPTH, _D, 128), jnp.float32),
            pltpu.VMEM((_D, _D), jnp.float32),
            pltpu.VMEM((_D, _D), jnp.float32),
            pltpu.VMEM((_L * _L,), jnp.float32),
            pltpu.VMEM((_BPW,), jnp.float32),
            pltpu.SemaphoreType.DMA,
            pltpu.SemaphoreType.DMA,
            pltpu.SemaphoreType.DMA,
            pltpu.SemaphoreType.DMA,
        ],
        compiler_params=pltpu.CompilerParams(needs_layout_passes=False,
                                             use_tc_tiling_on_sc=True),
    )
    return run(user, item, uT, vT, tu, tv)


# speedup vs baseline: 1.2384x; 1.0547x over previous
import jax
import jax.numpy as jnp
from jax import lax
from jax.experimental import pallas as pl
from jax.experimental.pallas import tpu as pltpu
from jax.experimental.pallas import tpu_sc as plsc

_NC = 2
_NS = 16
_NW = _NC * _NS
_L = 16
_D = 64
_B = 16384
_BPW = _B // _NW
_DU = 8
_DV = 4


_TAIL = 999936
_LASTC = 999808


def _sc_body(user_hbm, item_hbm, uT_hbm, vT_hbm, tu_hbm, tv_hbm, out_hbm,
             idx_u, idx_v, bu, bv, tail_u, tail_v, p_buf, out_v, *sems):
    wid = lax.axis_index("s") * _NC + lax.axis_index("c")
    base = wid * _BPW
    sems_u = sems[:_DU]
    sems_v = sems[_DU:]

    pltpu.sync_copy(user_hbm.at[pl.ds(base, _BPW)], idx_u)
    pltpu.sync_copy(item_hbm.at[pl.ds(base, _BPW)], idx_v)
    pltpu.sync_copy(tu_hbm, tail_u)
    pltpu.sync_copy(tv_hbm, tail_v)

    def issue_u(iu, slot):
        cu = pl.multiple_of(
            jnp.minimum(lax.shift_right_logical(iu, 7) * 128, _LASTC), 128)
        pltpu.async_copy(uT_hbm.at[:, pl.ds(cu, 128)], bu.at[slot], sems_u[slot])

    def issue_v(iv, slot):
        cv = pl.multiple_of(
            jnp.minimum(lax.shift_right_logical(iv, 7) * 128, _LASTC), 128)
        pltpu.async_copy(vT_hbm.at[:, pl.ds(cv, 128)], bv.at[slot], sems_v[slot])

    def drain_u(slot):
        pltpu.make_async_copy(uT_hbm.at[:, pl.ds(0, 128)], bu.at[slot],
                              sems_u[slot]).wait()

    def drain_v(slot):
        pltpu.make_async_copy(vT_hbm.at[:, pl.ds(0, 128)], bv.at[slot],
                              sems_v[slot]).wait()

    lane = lax.iota(jnp.int32, _L)
    col_idx = lane * _L
    d_vecs = [lane + g * _L for g in range(_D // _L)]

    iu0 = idx_u[pl.ds(0, _L)]
    iv0 = idx_v[pl.ds(0, _L)]
    for k in range(_DU):
        issue_u(iu0[k], k)
    for k in range(_DV):
        issue_v(iv0[k], k)

    def step(j, carry):
        p = j * _L
        iu_vec = idx_u[pl.ds(p, _L)]
        iv_vec = idx_v[pl.ds(p, _L)]
        pn = jnp.minimum(p + _L, _BPW - _L)
        iu_nxt = idx_u[pl.ds(pn, _L)]
        iv_nxt = idx_v[pl.ds(pn, _L)]
        for k in range(_L):
            slot_u = k % _DU
            slot_v = k % _DV
            drain_u(slot_u)
            drain_v(slot_v)
            iu = iu_vec[k]
            iv = iv_vec[k]
            lu = jnp.full((_L,), 0, jnp.int32) + lax.bitwise_and(iu, 127)
            lv = jnp.full((_L,), 0, jnp.int32) + lax.bitwise_and(iv, 127)
            tmu = jnp.full((_L,), iu >= _TAIL)
            tmv = jnp.full((_L,), iv >= _TAIL)
            tlu = jnp.full((_L,), 0, jnp.int32) + jnp.maximum(iu - _TAIL, 0)
            tlv = jnp.full((_L,), 0, jnp.int32) + jnp.maximum(iv - _TAIL, 0)
            acc = jnp.zeros((_L,), jnp.float32)
            for g in range(_D // _L):
                uvals = jnp.where(tmu,
                                  plsc.load_gather(tail_u, [d_vecs[g], tlu]),
                                  plsc.load_gather(bu.at[slot_u], [d_vecs[g], lu]))
                vvals = jnp.where(tmv,
                                  plsc.load_gather(tail_v, [d_vecs[g], tlv]),
                                  plsc.load_gather(bv.at[slot_v], [d_vecs[g], lv]))
                acc += uvals * vvals
            p_buf[pl.ds(k * _L, _L)] = acc
            au = k + _DU
            @pl.when(p + au < _BPW)
            def _():
                if au < _L:
                    issue_u(iu_vec[au], slot_u)
                else:
                    issue_u(iu_nxt[au - _L], slot_u)
            av = k + _DV
            @pl.when(p + av < _BPW)
            def _():
                if av < _L:
                    issue_v(iv_vec[av], slot_v)
                else:
                    issue_v(iv_nxt[av - _L], slot_v)
        tot = plsc.load_gather(p_buf, [col_idx])
        for t in range(1, _L):
            tot += plsc.load_gather(p_buf, [col_idx + t])
        out_v[pl.ds(p, _L)] = tot
        return carry

    lax.fori_loop(0, _BPW // _L, step, 0)

    pltpu.sync_copy(out_v, out_hbm.at[pl.ds(base, _BPW)])


@jax.jit
def kernel(user, item, user_factors, item_factors):
    uT = user_factors.T
    vT = item_factors.T
    tu = user_factors[_TAIL:].T
    tv = item_factors[_TAIL:].T
    mesh = plsc.VectorSubcoreMesh(core_axis_name="c", subcore_axis_name="s",
                                  num_cores=_NC, num_subcores=_NS)
    run = pl.kernel(
        _sc_body,
        out_type=jax.ShapeDtypeStruct((_B,), jnp.float32),
        mesh=mesh,
        scratch_types=[
            pltpu.VMEM((_BPW,), jnp.int32),
            pltpu.VMEM((_BPW,), jnp.int32),
            pltpu.VMEM((_DU, _D, 128), jnp.float32),
            pltpu.VMEM((_DV, _D, 128), jnp.float32),
            pltpu.VMEM((_D, _D), jnp.float32),
            pltpu.VMEM((_D, _D), jnp.float32),
            pltpu.VMEM((_L * _L,), jnp.float32),
            pltpu.VMEM((_BPW,), jnp.float32),
        ] + [pltpu.SemaphoreType.DMA] * (_DU + _DV),
        compiler_params=pltpu.CompilerParams(needs_layout_passes=False,
                                             use_tc_tiling_on_sc=True),
    )
    return run(user, item, uT, vT, tu, tv)
